# Initial kernel scaffold; baseline (speedup 1.0000x reference)
import jax
import jax.numpy as jnp
from jax.experimental import pallas as pl


def kernel(edge_index, edge_weight, batch_users, batch_pos_items, batch_neg_items, item_content, user_table, item_table, Wc):
    raise NotImplementedError("write your pallas kernel here")


# speedup vs baseline: 9.7990x; 9.7990x over previous
import functools

import jax
import jax.numpy as jnp
from jax import lax
from jax.experimental import pallas as pl
from jax.experimental.pallas import tpu as pltpu
from jax.experimental.pallas import tpu_sc as plsc

NU = 25000
NI = 25000
N = NU + NI
D = 64
DH = 32
NL = 3
E = 800000
B = 4096
NS = 16
NC = 2
EW = E // NS
K = 400
NCH = EW // K
RPW = 3200
RC = 80
DGR = 3136
DGC = 1568
F32 = jnp.float32
I32 = jnp.int32


def _sc_body(srcv, dstv, wv, x0f, buv, bpv, bnv,
             xmf, brf, x1f, x2f, x3f, nrmf,
             accum, degv,
             idxa, idxb, valv, gsa, gsb, rows, xbuf, xzero, dbuf, bidx,
             sem):
    c = lax.axis_index("c")
    s = lax.axis_index("s")
    coff = c * N

    def _row_loop(fn):
        cnt = jnp.where(s == 15, 25, 40)

        def _b(t, _):
            fn(s * RPW + t * RC)
            return 0
        lax.fori_loop(0, cnt, _b, 0)

    def _for_deg_chunks(fn):
        @pl.when(s < 15)
        def _():
            fn(s * DGR, DGC)
            fn(s * DGR + DGC, DGC)

        @pl.when(s == 15)
        def _():
            fn(15 * DGR, DGC)
            fn(15 * DGR + DGC, N - 15 * DGR - DGC)

    def _z16(i, _):
        dbuf[pl.ds(i * 16, 16)] = jnp.zeros((16,), F32)
        return 0
    lax.fori_loop(0, DGC // 16, _z16, 0)

    def _zrow(i, _):
        xzero[i, pl.ds(0, 16)] = jnp.zeros((16,), F32)
        xzero[i, pl.ds(16, 16)] = jnp.zeros((16,), F32)
        return 0
    lax.fori_loop(0, RC, _zrow, 0)

    _for_deg_chunks(lambda d0, sz: pltpu.sync_copy(
        dbuf.at[pl.ds(0, sz)], degv.at[pl.ds(d0, sz)]))
    plsc.subcore_barrier()

    def _deg_chunk(j, _):
        base = s * EW + j * K
        pltpu.sync_copy(dstv.at[pl.ds(base, K)], idxb)
        pltpu.sync_copy(wv.at[pl.ds(base, K)], valv)
        pltpu.sync_copy(valv, degv.at[idxb], add=True)
        return 0
    lax.fori_loop(0, NCH, _deg_chunk, 0)
    plsc.subcore_barrier()

    def _rsqrt_chunk(d0, sz):
        pltpu.sync_copy(degv.at[pl.ds(d0, sz)], dbuf.at[pl.ds(0, sz)])

        def _nr16(i, _):
            x = dbuf[pl.ds(i * 16, 16)]
            ii = plsc.bitcast(x, I32)
            ii = jnp.int32(0x5F3759DF) - lax.shift_right_logical(ii, 1)
            y = plsc.bitcast(ii, F32)
            for _u in range(3):
                y = y * (1.5 - 0.5 * x * y * y)
            dbuf[pl.ds(i * 16, 16)] = jnp.where(x > 0.0, y, 0.0)
            return 0
        lax.fori_loop(0, sz // 16, _nr16, 0)
        pltpu.sync_copy(dbuf.at[pl.ds(0, sz)], degv.at[pl.ds(d0, sz)])
    _for_deg_chunks(_rsqrt_chunk)
    plsc.subcore_barrier()

    def _norm_chunk(j, _):
        base = s * EW + j * K
        pltpu.sync_copy(srcv.at[pl.ds(base, K)], idxa)
        pltpu.sync_copy(dstv.at[pl.ds(base, K)], idxb)
        pltpu.sync_copy(wv.at[pl.ds(base, K)], valv)
        pltpu.async_copy(degv.at[idxa], gsa, sem).wait()
        pltpu.async_copy(degv.at[idxb], gsb, sem).wait()

        def _n16(g, _2):
            sl = pl.ds(g * 16, 16)
            valv[sl] = gsa[sl] * valv[sl] * gsb[sl]
            return 0
        lax.fori_loop(0, K // 16, _n16, 0)
        pltpu.sync_copy(valv, nrmf.at[pl.ds(c * E + base, K)])
        return 0
    lax.fori_loop(0, NCH, _norm_chunk, 0)

    def run_layer(xin, xout):
        _row_loop(lambda r0: pltpu.sync_copy(xzero, accum.at[pl.ds(r0, RC)]))
        plsc.subcore_barrier()

        def _echunk(j, _):
            base = s * EW + j * K
            pltpu.sync_copy(srcv.at[pl.ds(base, K)], idxa)
            pltpu.sync_copy(dstv.at[pl.ds(base, K)], idxb)
            pltpu.sync_copy(nrmf.at[pl.ds(c * E + base, K)], valv)

            def _off16(g, _2):
                sl = pl.ds(g * 16, 16)
                idxa[sl] = idxa[sl] + coff
                return 0
            lax.fori_loop(0, K // 16, _off16, 0)
            pltpu.async_copy(xin.at[idxa], rows, sem).wait()

            def _mul16(g, _2):
                nvec = valv[pl.ds(g * 16, 16)]
                for e in range(16):
                    idx = g * 16 + e
                    nb = jnp.full((16,), nvec[e], F32)
                    rows[idx, pl.ds(0, 16)] = rows[idx, pl.ds(0, 16)] * nb
                    rows[idx, pl.ds(16, 16)] = rows[idx, pl.ds(16, 16)] * nb
                return 0
            lax.fori_loop(0, K // 16, _mul16, 0)
            pltpu.sync_copy(rows, accum.at[idxb], add=True)
            return 0
        lax.fori_loop(0, NCH, _echunk, 0)
        plsc.subcore_barrier()

        def _wb(r0):
            pltpu.sync_copy(accum.at[pl.ds(r0, RC)], xbuf)
            pltpu.sync_copy(xbuf, xout.at[pl.ds(coff + r0, RC)])
        _row_loop(_wb)
        plsc.subcore_barrier()

    run_layer(x0f, x1f)
    run_layer(x1f, x2f)
    run_layer(x2f, x3f)

    def _mean(r0):
        pltpu.sync_copy(x0f.at[pl.ds(coff + r0, RC)], xbuf)
        for xl in (x1f, x2f, x3f):
            pltpu.sync_copy(xl.at[pl.ds(coff + r0, RC)], xzero)

            def _acc(i, _):
                for h in (0, 16):
                    sl = pl.ds(h, 16)
                    xbuf[i, sl] = xbuf[i, sl] + xzero[i, sl]
                return 0
            lax.fori_loop(0, RC, _acc, 0)

        def _scale(i, _):
            for h in (0, 16):
                sl = pl.ds(h, 16)
                xbuf[i, sl] = xbuf[i, sl] * 0.25
            return 0
        lax.fori_loop(0, RC, _scale, 0)
        pltpu.sync_copy(xbuf, xmf.at[pl.ds(coff + r0, RC)])
    _row_loop(_mean)
    plsc.subcore_barrier()


    BPW = B // NS
    for q, (bref, noff) in enumerate(((buv, 0), (bpv, NU), (bnv, NU))):
        b0 = s * BPW
        pltpu.sync_copy(bref.at[pl.ds(b0, BPW)], bidx)
        off = coff + noff

        def _boff(g, _):
            sl = pl.ds(g * 16, 16)
            bidx[sl] = bidx[sl] + off
            return 0
        lax.fori_loop(0, BPW // 16, _boff, 0)
        for t in range(BPW // 64):
            pltpu.async_copy(xmf.at[bidx.at[pl.ds(t * 64, 64)]],
                             xbuf.at[pl.ds(0, 64)], sem).wait()
            pltpu.sync_copy(
                xbuf.at[pl.ds(0, 64)],
                brf.at[pl.ds(q * 2 * B + c * B + b0 + t * 64, 64)])


def _sc_propagate(srcv, dstv, wv, x0f, buv, bpv, bnv):
    mesh = plsc.VectorSubcoreMesh(core_axis_name="c", subcore_axis_name="s")
    f = functools.partial(
        pl.kernel,
        out_type=(
            jax.ShapeDtypeStruct((NC * N, DH), F32),
            jax.ShapeDtypeStruct((3 * NC * B, DH), F32),
            jax.ShapeDtypeStruct((NC * N, DH), F32),
            jax.ShapeDtypeStruct((NC * N, DH), F32),
            jax.ShapeDtypeStruct((NC * N, DH), F32),
            jax.ShapeDtypeStruct((NC * E,), F32),
        ),
        mesh=mesh,
        compiler_params=pltpu.CompilerParams(use_tc_tiling_on_sc=False,
                                             needs_layout_passes=False),
        scratch_types=(
            pltpu.VMEM_SHARED((N, DH), F32),
            pltpu.VMEM_SHARED((N,), F32),
            pltpu.VMEM((K,), I32),
            pltpu.VMEM((K,), I32),
            pltpu.VMEM((K,), F32),
            pltpu.VMEM((K,), F32),
            pltpu.VMEM((K,), F32),
            pltpu.VMEM((K, DH), F32),
            pltpu.VMEM((RC, DH), F32),
            pltpu.VMEM((RC, DH), F32),
            pltpu.VMEM((DGC,), F32),
            pltpu.VMEM((B // NS,), I32),
            pltpu.SemaphoreType.DMA,
        ),
    )(_sc_body)
    return f(srcv, dstv, wv, x0f, buv, bpv, bnv)


_AR = 1000


def _align_body(cont, wc, ifin, acc):
    i = pl.program_id(0)

    @pl.when(i == 0)
    def _():
        acc[...] = jnp.zeros((1, 1), F32)

    proj = lax.dot_general(cont[...], wc[...],
                           (((1,), (1,)), ((), ())),
                           preferred_element_type=F32)
    d = ifin[...] - proj
    acc[...] += jnp.sum(d * d).reshape(1, 1)


def _bpr_body(u, ip, ing, bpr, reg):
    uv = u[...]
    pv = ip[...]
    nv = ing[...]
    dsc = jnp.sum(uv * pv, axis=1) - jnp.sum(uv * nv, axis=1)
    x = -dsc
    m = jnp.maximum(x, 0.0)
    sp = m + jnp.log(jnp.exp(x - m) + jnp.exp(-m))
    bpr[...] = jnp.sum(sp).reshape(1, 1)
    reg[...] = (jnp.sum(uv * uv) + jnp.sum(pv * pv)
                + jnp.sum(nv * nv)).reshape(1, 1)


def kernel(edge_index, edge_weight, batch_users, batch_pos_items,
           batch_neg_items, item_content, user_table, item_table, Wc):
    srcv = edge_index[0].astype(I32)
    dstv = edge_index[1].astype(I32)
    wv = edge_weight.astype(F32)

    x0 = jnp.concatenate([user_table, item_table], axis=0)
    x0f = x0.reshape(N, NC, DH).transpose(1, 0, 2).reshape(NC * N, DH)

    xmf, brf, _x1, _x2, _x3, _nrm = _sc_propagate(
        srcv, dstv, wv, x0f,
        batch_users.astype(I32), batch_pos_items.astype(I32),
        batch_neg_items.astype(I32))

    xm = xmf.reshape(NC, N, DH).transpose(1, 0, 2).reshape(N, D)
    u_final = xm[:NU]
    i_final = xm[NU:]

    br = brf.reshape(3, NC, B, DH).transpose(0, 2, 1, 3).reshape(3, B, D)
    u, ipos, ineg = br[0], br[1], br[2]

    align = pl.pallas_call(
        _align_body,
        grid=(NI // _AR,),
        in_specs=[
            pl.BlockSpec((_AR, 256), lambda i: (i, 0)),
            pl.BlockSpec((D, 256), lambda i: (0, 0)),
            pl.BlockSpec((_AR, D), lambda i: (i, 0)),
        ],
        out_specs=pl.BlockSpec((1, 1), lambda i: (0, 0)),
        out_shape=jax.ShapeDtypeStruct((1, 1), F32),
    )(item_content, Wc, i_final)

    bpr_s, reg_s = pl.pallas_call(
        _bpr_body,
        out_shape=(jax.ShapeDtypeStruct((1, 1), F32),
                   jax.ShapeDtypeStruct((1, 1), F32)),
    )(u, ipos, ineg)

    loss = (bpr_s[0, 0] / B + 1e-4 * (reg_s[0, 0] / B)
            + 0.1 * (align[0, 0] / (NI * D)))
    return loss, u_final, i_final


# speedup vs baseline: 14.6224x; 1.4922x over previous
import functools

import jax
import jax.numpy as jnp
from jax import lax
from jax.experimental import pallas as pl
from jax.experimental.pallas import tpu as pltpu
from jax.experimental.pallas import tpu_sc as plsc

NU = 25000
NI = 25000
N = NU + NI
D = 64
DH = 32
NL = 3
E = 800000
B = 4096
NS = 16
NC = 2
EW = E // NS
K = 400
NCH = EW // K
RPW = 3200
RC = 80
DGR = 3136
DGC = 1568
F32 = jnp.float32
I32 = jnp.int32


def _sc_body(srcv, dstv, wv, x0f, buv, bpv, bnv,
             xmf, brf, x1f, x2f, x3f, nrmf,
             accum, degv,
             idxa, idxb, valv, idxa2, idxb2, valv2,
             gsa, gsb, rows, xbuf, xzero, dbuf, bidx,
             sem, sem_pf0, sem_pf1, sem_d, sem_n):
    c = lax.axis_index("c")
    s = lax.axis_index("s")
    coff = c * N

    def _row_loop(fn):
        cnt = jnp.where(s == 15, 25, 40)

        def _b(t, _):
            fn(s * RPW + t * RC)
            return 0
        lax.fori_loop(0, cnt, _b, 0)

    def _for_deg_chunks(fn):
        @pl.when(s < 15)
        def _():
            fn(s * DGR, DGC)
            fn(s * DGR + DGC, DGC)

        @pl.when(s == 15)
        def _():
            fn(15 * DGR, DGC)
            fn(15 * DGR + DGC, N - 15 * DGR - DGC)

    def _z16(i, _):
        dbuf[pl.ds(i * 16, 16)] = jnp.zeros((16,), F32)
        return 0
    lax.fori_loop(0, DGC // 16, _z16, 0)

    def _zrow(i, _):
        xzero[i, pl.ds(0, 16)] = jnp.zeros((16,), F32)
        xzero[i, pl.ds(16, 16)] = jnp.zeros((16,), F32)
        return 0
    lax.fori_loop(0, RC, _zrow, 0)

    _for_deg_chunks(lambda d0, sz: pltpu.sync_copy(
        dbuf.at[pl.ds(0, sz)], degv.at[pl.ds(d0, sz)]))
    plsc.subcore_barrier()

    def _deg_issue(j, ib, va, sm):
        base = s * EW + j * K
        return (pltpu.async_copy(dstv.at[pl.ds(base, K)], ib, sm),
                pltpu.async_copy(wv.at[pl.ds(base, K)], va, sm))

    def _deg_pair(m, _):
        da = _deg_issue(2 * m, idxb, valv, sem_pf0)
        db = _deg_issue(2 * m + 1, idxb2, valv2, sem_pf1)
        for d in da:
            d.wait()
        pltpu.sync_copy(valv, degv.at[idxb], add=True)
        for d in db:
            d.wait()
        pltpu.sync_copy(valv2, degv.at[idxb2], add=True)
        return 0
    lax.fori_loop(0, NCH // 2, _deg_pair, 0)
    if NCH % 2:
        da = _deg_issue(NCH - 1, idxb, valv, sem_pf0)
        for d in da:
            d.wait()
        pltpu.sync_copy(valv, degv.at[idxb], add=True)
    plsc.subcore_barrier()

    def _rsqrt_chunk(d0, sz):
        pltpu.sync_copy(degv.at[pl.ds(d0, sz)], dbuf.at[pl.ds(0, sz)])

        def _nr16(i, _):
            x = dbuf[pl.ds(i * 16, 16)]
            ii = plsc.bitcast(x, I32)
            ii = jnp.int32(0x5F3759DF) - lax.shift_right_logical(ii, 1)
            y = plsc.bitcast(ii, F32)
            for _u in range(3):
                y = y * (1.5 - 0.5 * x * y * y)
            dbuf[pl.ds(i * 16, 16)] = jnp.where(x > 0.0, y, 0.0)
            return 0
        lax.fori_loop(0, sz // 16, _nr16, 0)
        pltpu.sync_copy(dbuf.at[pl.ds(0, sz)], degv.at[pl.ds(d0, sz)])
    _for_deg_chunks(_rsqrt_chunk)
    plsc.subcore_barrier()

    def run_layer(xin, xout, first):
        _row_loop(lambda r0: pltpu.sync_copy(xzero, accum.at[pl.ds(r0, RC)]))
        plsc.subcore_barrier()

        def _issue(j, ia, ib, va, sm):
            base = s * EW + j * K
            ds_ = [pltpu.async_copy(srcv.at[pl.ds(base, K)], ia, sm),
                   pltpu.async_copy(dstv.at[pl.ds(base, K)], ib, sm)]
            if first:
                ds_.append(pltpu.async_copy(wv.at[pl.ds(base, K)], va, sm))
            else:
                ds_.append(pltpu.async_copy(
                    nrmf.at[pl.ds(c * E + base, K)], va, sm))
            return ds_

        def _process(j, ia, ib, va):
            if first:
                d1 = pltpu.async_copy(degv.at[ia], gsa, sem_d)
                d2 = pltpu.async_copy(degv.at[ib], gsb, sem_d)
                d1.wait()
                d2.wait()

            def _off16(g, _2):
                sl = pl.ds(g * 16, 16)
                ia[sl] = ia[sl] + coff
                return 0
            lax.fori_loop(0, K // 16, _off16, 0)
            dg = pltpu.async_copy(xin.at[ia], rows, sem)
            if first:
                def _n16(g, _2):
                    sl = pl.ds(g * 16, 16)
                    va[sl] = gsa[sl] * va[sl] * gsb[sl]
                    return 0
                lax.fori_loop(0, K // 16, _n16, 0)
                base = s * EW + j * K
                dn = pltpu.async_copy(va, nrmf.at[pl.ds(c * E + base, K)],
                                      sem_n)
            dg.wait()

            def _mul16(g, _2):
                nvec = va[pl.ds(g * 16, 16)]
                for e in range(16):
                    idx = g * 16 + e
                    nb = jnp.full((16,), nvec[e], F32)
                    rows[idx, pl.ds(0, 16)] = rows[idx, pl.ds(0, 16)] * nb
                    rows[idx, pl.ds(16, 16)] = rows[idx, pl.ds(16, 16)] * nb
                return 0
            lax.fori_loop(0, K // 16, _mul16, 0)
            if first:
                dn.wait()
            pltpu.sync_copy(rows, accum.at[ib], add=True)

        def _pair(m, _):
            da = _issue(2 * m, idxa, idxb, valv, sem_pf0)
            db = _issue(2 * m + 1, idxa2, idxb2, valv2, sem_pf1)
            for d in da:
                d.wait()
            _process(2 * m, idxa, idxb, valv)
            for d in db:
                d.wait()
            _process(2 * m + 1, idxa2, idxb2, valv2)
            return 0
        lax.fori_loop(0, NCH // 2, _pair, 0)
        if NCH % 2:
            da = _issue(NCH - 1, idxa, idxb, valv, sem_pf0)
            for d in da:
                d.wait()
            _process(NCH - 1, idxa, idxb, valv)
        plsc.subcore_barrier()

        def _wb(r0):
            pltpu.sync_copy(accum.at[pl.ds(r0, RC)], xbuf)
            pltpu.sync_copy(xbuf, xout.at[pl.ds(coff + r0, RC)])
        _row_loop(_wb)
        plsc.subcore_barrier()

    run_layer(x0f, x1f, True)
    run_layer(x1f, x2f, False)
    run_layer(x2f, x3f, False)

    def _mean(r0):
        pltpu.sync_copy(x0f.at[pl.ds(coff + r0, RC)], xbuf)
        for xl in (x1f, x2f, x3f):
            pltpu.sync_copy(xl.at[pl.ds(coff + r0, RC)], xzero)

            def _acc(i, _):
                for h in (0, 16):
                    sl = pl.ds(h, 16)
                    xbuf[i, sl] = xbuf[i, sl] + xzero[i, sl]
                return 0
            lax.fori_loop(0, RC, _acc, 0)

        def _scale(i, _):
            for h in (0, 16):
                sl = pl.ds(h, 16)
                xbuf[i, sl] = xbuf[i, sl] * 0.25
            return 0
        lax.fori_loop(0, RC, _scale, 0)
        pltpu.sync_copy(xbuf, xmf.at[pl.ds(coff + r0, RC)])
    _row_loop(_mean)
    plsc.subcore_barrier()


    BPW = B // NS
    for q, (bref, noff) in enumerate(((buv, 0), (bpv, NU), (bnv, NU))):
        b0 = s * BPW
        pltpu.sync_copy(bref.at[pl.ds(b0, BPW)], bidx)
        off = coff + noff

        def _boff(g, _):
            sl = pl.ds(g * 16, 16)
            bidx[sl] = bidx[sl] + off
            return 0
        lax.fori_loop(0, BPW // 16, _boff, 0)
        for t in range(BPW // 64):
            pltpu.async_copy(xmf.at[bidx.at[pl.ds(t * 64, 64)]],
                             xbuf.at[pl.ds(0, 64)], sem).wait()
            pltpu.sync_copy(
                xbuf.at[pl.ds(0, 64)],
                brf.at[pl.ds(q * 2 * B + c * B + b0 + t * 64, 64)])


def _sc_propagate(srcv, dstv, wv, x0f, buv, bpv, bnv):
    mesh = plsc.VectorSubcoreMesh(core_axis_name="c", subcore_axis_name="s")
    f = functools.partial(
        pl.kernel,
        out_type=(
            jax.ShapeDtypeStruct((NC * N, DH), F32),
            jax.ShapeDtypeStruct((3 * NC * B, DH), F32),
            jax.ShapeDtypeStruct((NC * N, DH), F32),
            jax.ShapeDtypeStruct((NC * N, DH), F32),
            jax.ShapeDtypeStruct((NC * N, DH), F32),
            jax.ShapeDtypeStruct((NC * E,), F32),
        ),
        mesh=mesh,
        compiler_params=pltpu.CompilerParams(use_tc_tiling_on_sc=False,
                                             needs_layout_passes=False),
        scratch_types=(
            pltpu.VMEM_SHARED((N, DH), F32),
            pltpu.VMEM_SHARED((N,), F32),
            pltpu.VMEM((K,), I32),
            pltpu.VMEM((K,), I32),
            pltpu.VMEM((K,), F32),
            pltpu.VMEM((K,), I32),
            pltpu.VMEM((K,), I32),
            pltpu.VMEM((K,), F32),
            pltpu.VMEM((K,), F32),
            pltpu.VMEM((K,), F32),
            pltpu.VMEM((K, DH), F32),
            pltpu.VMEM((RC, DH), F32),
            pltpu.VMEM((RC, DH), F32),
            pltpu.VMEM((DGC,), F32),
            pltpu.VMEM((B // NS,), I32),
            pltpu.SemaphoreType.DMA,
            pltpu.SemaphoreType.DMA,
            pltpu.SemaphoreType.DMA,
            pltpu.SemaphoreType.DMA,
            pltpu.SemaphoreType.DMA,
        ),
    )(_sc_body)
    return f(srcv, dstv, wv, x0f, buv, bpv, bnv)


_AR = 1000


def _align_body(cont, wc, ifin, acc):
    i = pl.program_id(0)

    @pl.when(i == 0)
    def _():
        acc[...] = jnp.zeros((1, 1), F32)

    proj = lax.dot_general(cont[...], wc[...],
                           (((1,), (1,)), ((), ())),
                           preferred_element_type=F32)
    d = ifin[...] - proj
    acc[...] += jnp.sum(d * d).reshape(1, 1)


def _bpr_body(u, ip, ing, bpr, reg):
    uv = u[...]
    pv = ip[...]
    nv = ing[...]
    dsc = jnp.sum(uv * pv, axis=1) - jnp.sum(uv * nv, axis=1)
    x = -dsc
    m = jnp.maximum(x, 0.0)
    sp = m + jnp.log(jnp.exp(x - m) + jnp.exp(-m))
    bpr[...] = jnp.sum(sp).reshape(1, 1)
    reg[...] = (jnp.sum(uv * uv) + jnp.sum(pv * pv)
                + jnp.sum(nv * nv)).reshape(1, 1)


def kernel(edge_index, edge_weight, batch_users, batch_pos_items,
           batch_neg_items, item_content, user_table, item_table, Wc):
    srcv = edge_index[0].astype(I32)
    dstv = edge_index[1].astype(I32)
    wv = edge_weight.astype(F32)

    x0 = jnp.concatenate([user_table, item_table], axis=0)
    x0f = x0.reshape(N, NC, DH).transpose(1, 0, 2).reshape(NC * N, DH)

    xmf, brf, _x1, _x2, _x3, _nrm = _sc_propagate(
        srcv, dstv, wv, x0f,
        batch_users.astype(I32), batch_pos_items.astype(I32),
        batch_neg_items.astype(I32))

    xm = xmf.reshape(NC, N, DH).transpose(1, 0, 2).reshape(N, D)
    u_final = xm[:NU]
    i_final = xm[NU:]

    br = brf.reshape(3, NC, B, DH).transpose(0, 2, 1, 3).reshape(3, B, D)
    u, ipos, ineg = br[0], br[1], br[2]

    align = pl.pallas_call(
        _align_body,
        grid=(NI // _AR,),
        in_specs=[
            pl.BlockSpec((_AR, 256), lambda i: (i, 0)),
            pl.BlockSpec((D, 256), lambda i: (0, 0)),
            pl.BlockSpec((_AR, D), lambda i: (i, 0)),
        ],
        out_specs=pl.BlockSpec((1, 1), lambda i: (0, 0)),
        out_shape=jax.ShapeDtypeStruct((1, 1), F32),
    )(item_content, Wc, i_final)

    bpr_s, reg_s = pl.pallas_call(
        _bpr_body,
        out_shape=(jax.ShapeDtypeStruct((1, 1), F32),
                   jax.ShapeDtypeStruct((1, 1), F32)),
    )(u, ipos, ineg)

    loss = (bpr_s[0, 0] / B + 1e-4 * (reg_s[0, 0] / B)
            + 0.1 * (align[0, 0] / (NI * D)))
    return loss, u_final, i_final


# speedup vs baseline: 16.8114x; 1.1497x over previous
import functools

import jax
import jax.numpy as jnp
from jax import lax
from jax.experimental import pallas as pl
from jax.experimental.pallas import tpu as pltpu
from jax.experimental.pallas import tpu_sc as plsc

NU = 25000
NI = 25000
N = NU + NI
D = 64
DH = 32
NL = 3
E = 800000
B = 4096
NS = 16
NC = 2
EW = E // NS
K = 400
K2 = K // 2
NCH = EW // K
RPW = 3200
RC = 80
DGR = 3136
DGC = 1568
F32 = jnp.float32
I32 = jnp.int32


def _sc_body(srcv, dstv, wv, x0f, buv, bpv, bnv,
             xmf, brf, x1f, x2f, x3f, nrmf,
             accum, degv,
             idxa, ibl, ibh, valv, idxa2, ibl2, ibh2, valv2,
             gsa, gsb, rows, xbuf, xzero, xzero2, dbuf, bidx,
             sem, sem2, sem_pf0, sem_pf1, sem_d, sem_n, sem_s):
    c = lax.axis_index("c")
    s = lax.axis_index("s")
    coff = c * N

    def _row_loop(fn):
        cnt = jnp.where(s == 15, 25, 40)

        def _b(t, _):
            fn(s * RPW + t * RC)
            return 0
        lax.fori_loop(0, cnt, _b, 0)

    def _for_deg_chunks(fn):
        @pl.when(s < 15)
        def _():
            fn(s * DGR, DGC)
            fn(s * DGR + DGC, DGC)

        @pl.when(s == 15)
        def _():
            fn(15 * DGR, DGC)
            fn(15 * DGR + DGC, N - 15 * DGR - DGC)

    def _z16(i, _):
        dbuf[pl.ds(i * 16, 16)] = jnp.zeros((16,), F32)
        return 0
    lax.fori_loop(0, DGC // 16, _z16, 0)

    def _zrow(i, _):
        xzero[i, pl.ds(0, 16)] = jnp.zeros((16,), F32)
        xzero[i, pl.ds(16, 16)] = jnp.zeros((16,), F32)
        return 0
    lax.fori_loop(0, RC, _zrow, 0)

    _for_deg_chunks(lambda d0, sz: pltpu.sync_copy(
        dbuf.at[pl.ds(0, sz)], degv.at[pl.ds(d0, sz)]))
    plsc.subcore_barrier()

    def _deg_issue(j, bl, bh, va, sm):
        base = s * EW + j * K
        return (pltpu.async_copy(dstv.at[pl.ds(base, K2)], bl, sm),
                pltpu.async_copy(dstv.at[pl.ds(base + K2, K2)], bh, sm),
                pltpu.async_copy(wv.at[pl.ds(base, K)], va, sm))

    def _deg_scatter(bl, bh, va):
        d0 = pltpu.async_copy(va.at[pl.ds(0, K2)], degv.at[bl], sem_s,
                              add=True)
        pltpu.sync_copy(va.at[pl.ds(K2, K2)], degv.at[bh], add=True)
        d0.wait()

    def _deg_pair(m, _):
        da = _deg_issue(2 * m, ibl, ibh, valv, sem_pf0)
        db = _deg_issue(2 * m + 1, ibl2, ibh2, valv2, sem_pf1)
        for d in da:
            d.wait()
        _deg_scatter(ibl, ibh, valv)
        for d in db:
            d.wait()
        _deg_scatter(ibl2, ibh2, valv2)
        return 0
    lax.fori_loop(0, NCH // 2, _deg_pair, 0)
    if NCH % 2:
        da = _deg_issue(NCH - 1, ibl, ibh, valv, sem_pf0)
        for d in da:
            d.wait()
        _deg_scatter(ibl, ibh, valv)
    plsc.subcore_barrier()

    def _rsqrt_chunk(d0, sz):
        pltpu.sync_copy(degv.at[pl.ds(d0, sz)], dbuf.at[pl.ds(0, sz)])

        def _nr16(i, _):
            x = dbuf[pl.ds(i * 16, 16)]
            ii = plsc.bitcast(x, I32)
            ii = jnp.int32(0x5F3759DF) - lax.shift_right_logical(ii, 1)
            y = plsc.bitcast(ii, F32)
            for _u in range(3):
                y = y * (1.5 - 0.5 * x * y * y)
            dbuf[pl.ds(i * 16, 16)] = jnp.where(x > 0.0, y, 0.0)
            return 0
        lax.fori_loop(0, sz // 16, _nr16, 0)
        pltpu.sync_copy(dbuf.at[pl.ds(0, sz)], degv.at[pl.ds(d0, sz)])
    _for_deg_chunks(_rsqrt_chunk)
    plsc.subcore_barrier()

    def run_layer(xin, xout, first):
        _row_loop(lambda r0: pltpu.sync_copy(xzero, accum.at[pl.ds(r0, RC)]))
        plsc.subcore_barrier()

        def _issue(j, ia, bl, bh, va, sm):
            base = s * EW + j * K
            ds_ = [pltpu.async_copy(srcv.at[pl.ds(base, K)], ia, sm),
                   pltpu.async_copy(dstv.at[pl.ds(base, K2)], bl, sm),
                   pltpu.async_copy(dstv.at[pl.ds(base + K2, K2)], bh, sm)]
            if first:
                ds_.append(pltpu.async_copy(wv.at[pl.ds(base, K)], va, sm))
            else:
                ds_.append(pltpu.async_copy(
                    nrmf.at[pl.ds(c * E + base, K)], va, sm))
            return ds_

        def _mul16(va, h):
            def _b(g, _2):
                e0 = h * K2 + g * 16
                nvec = va[pl.ds(e0, 16)]
                for e in range(16):
                    idx = e0 + e
                    nb = jnp.full((16,), nvec[e], F32)
                    rows[idx, pl.ds(0, 16)] = rows[idx, pl.ds(0, 16)] * nb
                    rows[idx, pl.ds(16, 16)] = rows[idx, pl.ds(16, 16)] * nb
                return 0
            lax.fori_loop(0, K2 // 16, _b, 0)

        def _process(j, ia, bl, bh, va):
            if first:
                d1 = pltpu.async_copy(degv.at[ia], gsa, sem_d)
                d2 = pltpu.async_copy(degv.at[bl], gsb.at[pl.ds(0, K2)],
                                      sem_d)
                d3 = pltpu.async_copy(degv.at[bh], gsb.at[pl.ds(K2, K2)],
                                      sem_d)
                d1.wait()
                d2.wait()
                d3.wait()

            def _off16(g, _2):
                sl = pl.ds(g * 16, 16)
                ia[sl] = ia[sl] + coff
                return 0
            lax.fori_loop(0, K // 16, _off16, 0)
            dg0 = pltpu.async_copy(xin.at[ia.at[pl.ds(0, K2)]],
                                   rows.at[pl.ds(0, K2)], sem)
            dg1 = pltpu.async_copy(xin.at[ia.at[pl.ds(K2, K2)]],
                                   rows.at[pl.ds(K2, K2)], sem2)
            if first:
                def _n16(g, _2):
                    sl = pl.ds(g * 16, 16)
                    va[sl] = gsa[sl] * va[sl] * gsb[sl]
                    return 0
                lax.fori_loop(0, K // 16, _n16, 0)
                base = s * EW + j * K
                dn = pltpu.async_copy(va, nrmf.at[pl.ds(c * E + base, K)],
                                      sem_n)
            dg0.wait()
            _mul16(va, 0)
            ds0 = pltpu.async_copy(rows.at[pl.ds(0, K2)], accum.at[bl],
                                   sem_s, add=True)
            dg1.wait()
            _mul16(va, 1)
            if first:
                dn.wait()
            pltpu.sync_copy(rows.at[pl.ds(K2, K2)], accum.at[bh], add=True)
            ds0.wait()

        def _pair(m, _):
            da = _issue(2 * m, idxa, ibl, ibh, valv, sem_pf0)
            db = _issue(2 * m + 1, idxa2, ibl2, ibh2, valv2, sem_pf1)
            for d in da:
                d.wait()
            _process(2 * m, idxa, ibl, ibh, valv)
            for d in db:
                d.wait()
            _process(2 * m + 1, idxa2, ibl2, ibh2, valv2)
            return 0
        lax.fori_loop(0, NCH // 2, _pair, 0)
        if NCH % 2:
            da = _issue(NCH - 1, idxa, ibl, ibh, valv, sem_pf0)
            for d in da:
                d.wait()
            _process(NCH - 1, idxa, ibl, ibh, valv)
        plsc.subcore_barrier()

        def _wb(r0):
            pltpu.sync_copy(accum.at[pl.ds(r0, RC)], xbuf)
            pltpu.sync_copy(xbuf, xout.at[pl.ds(coff + r0, RC)])
        _row_loop(_wb)
        plsc.subcore_barrier()

    run_layer(x0f, x1f, True)
    run_layer(x1f, x2f, False)
    run_layer(x2f, x3f, False)

    def _mean(r0):
        d0 = pltpu.async_copy(x0f.at[pl.ds(coff + r0, RC)], xbuf, sem)
        d1 = pltpu.async_copy(x1f.at[pl.ds(coff + r0, RC)], xzero, sem2)
        d2 = pltpu.async_copy(x2f.at[pl.ds(coff + r0, RC)], xzero2, sem_s)

        def _add(src_ref, scale):
            def _b(i, _):
                for h in (0, 16):
                    sl = pl.ds(h, 16)
                    v = xbuf[i, sl] + src_ref[i, sl]
                    xbuf[i, sl] = v * scale if scale != 1.0 else v
                return 0
            lax.fori_loop(0, RC, _b, 0)

        d0.wait()
        d1.wait()
        _add(xzero, 1.0)
        d2.wait()
        d3 = pltpu.async_copy(x3f.at[pl.ds(coff + r0, RC)], xzero, sem2)
        _add(xzero2, 1.0)
        d3.wait()
        _add(xzero, 0.25)
        pltpu.sync_copy(xbuf, xmf.at[pl.ds(coff + r0, RC)])
    _row_loop(_mean)
    plsc.subcore_barrier()

    BPW = B // NS
    for q, (bref, noff) in enumerate(((buv, 0), (bpv, NU), (bnv, NU))):
        b0 = s * BPW
        pltpu.sync_copy(bref.at[pl.ds(b0, BPW)], bidx)
        off = coff + noff

        def _boff(g, _):
            sl = pl.ds(g * 16, 16)
            bidx[sl] = bidx[sl] + off
            return 0
        lax.fori_loop(0, BPW // 16, _boff, 0)
        for t in range(BPW // 64):
            pltpu.async_copy(xmf.at[bidx.at[pl.ds(t * 64, 64)]],
                             xbuf.at[pl.ds(0, 64)], sem).wait()
            pltpu.sync_copy(
                xbuf.at[pl.ds(0, 64)],
                brf.at[pl.ds(q * 2 * B + c * B + b0 + t * 64, 64)])


def _sc_propagate(srcv, dstv, wv, x0f, buv, bpv, bnv):
    mesh = plsc.VectorSubcoreMesh(core_axis_name="c", subcore_axis_name="s")
    f = functools.partial(
        pl.kernel,
        out_type=(
            jax.ShapeDtypeStruct((NC * N, DH), F32),
            jax.ShapeDtypeStruct((3 * NC * B, DH), F32),
            jax.ShapeDtypeStruct((NC * N, DH), F32),
            jax.ShapeDtypeStruct((NC * N, DH), F32),
            jax.ShapeDtypeStruct((NC * N, DH), F32),
            jax.ShapeDtypeStruct((NC * E,), F32),
        ),
        mesh=mesh,
        compiler_params=pltpu.CompilerParams(use_tc_tiling_on_sc=False,
                                             needs_layout_passes=False),
        scratch_types=(
            pltpu.VMEM_SHARED((N, DH), F32),
            pltpu.VMEM_SHARED((N,), F32),
            pltpu.VMEM((K,), I32),
            pltpu.VMEM((K2,), I32),
            pltpu.VMEM((K2,), I32),
            pltpu.VMEM((K,), F32),
            pltpu.VMEM((K,), I32),
            pltpu.VMEM((K2,), I32),
            pltpu.VMEM((K2,), I32),
            pltpu.VMEM((K,), F32),
            pltpu.VMEM((K,), F32),
            pltpu.VMEM((K,), F32),
            pltpu.VMEM((K, DH), F32),
            pltpu.VMEM((RC, DH), F32),
            pltpu.VMEM((RC, DH), F32),
            pltpu.VMEM((RC, DH), F32),
            pltpu.VMEM((DGC,), F32),
            pltpu.VMEM((B // NS,), I32),
            pltpu.SemaphoreType.DMA,
            pltpu.SemaphoreType.DMA,
            pltpu.SemaphoreType.DMA,
            pltpu.SemaphoreType.DMA,
            pltpu.SemaphoreType.DMA,
            pltpu.SemaphoreType.DMA,
            pltpu.SemaphoreType.DMA,
        ),
    )(_sc_body)
    return f(srcv, dstv, wv, x0f, buv, bpv, bnv)


_AR = 1000


def _align_body(cont, wc, ifin, acc):
    i = pl.program_id(0)

    @pl.when(i == 0)
    def _():
        acc[...] = jnp.zeros((1, 1), F32)

    proj = lax.dot_general(cont[...], wc[...],
                           (((1,), (1,)), ((), ())),
                           preferred_element_type=F32)
    d = ifin[...] - proj
    acc[...] += jnp.sum(d * d).reshape(1, 1)


def _bpr_body(u, ip, ing, bpr, reg):
    uv = u[...]
    pv = ip[...]
    nv = ing[...]
    dsc = jnp.sum(uv * pv, axis=1) - jnp.sum(uv * nv, axis=1)
    x = -dsc
    m = jnp.maximum(x, 0.0)
    sp = m + jnp.log(jnp.exp(x - m) + jnp.exp(-m))
    bpr[...] = jnp.sum(sp).reshape(1, 1)
    reg[...] = (jnp.sum(uv * uv) + jnp.sum(pv * pv)
                + jnp.sum(nv * nv)).reshape(1, 1)


def kernel(edge_index, edge_weight, batch_users, batch_pos_items,
           batch_neg_items, item_content, user_table, item_table, Wc):
    srcv = edge_index[0].astype(I32)
    dstv = edge_index[1].astype(I32)
    wv = edge_weight.astype(F32)

    x0 = jnp.concatenate([user_table, item_table], axis=0)
    x0f = x0.reshape(N, NC, DH).transpose(1, 0, 2).reshape(NC * N, DH)

    xmf, brf, _x1, _x2, _x3, _nrm = _sc_propagate(
        srcv, dstv, wv, x0f,
        batch_users.astype(I32), batch_pos_items.astype(I32),
        batch_neg_items.astype(I32))

    xm = xmf.reshape(NC, N, DH).transpose(1, 0, 2).reshape(N, D)
    u_final = xm[:NU]
    i_final = xm[NU:]

    br = brf.reshape(3, NC, B, DH).transpose(0, 2, 1, 3).reshape(3, B, D)
    u, ipos, ineg = br[0], br[1], br[2]

    align = pl.pallas_call(
        _align_body,
        grid=(NI // _AR,),
        in_specs=[
            pl.BlockSpec((_AR, 256), lambda i: (i, 0)),
            pl.BlockSpec((D, 256), lambda i: (0, 0)),
            pl.BlockSpec((_AR, D), lambda i: (i, 0)),
        ],
        out_specs=pl.BlockSpec((1, 1), lambda i: (0, 0)),
        out_shape=jax.ShapeDtypeStruct((1, 1), F32),
    )(item_content, Wc, i_final)

    bpr_s, reg_s = pl.pallas_call(
        _bpr_body,
        out_shape=(jax.ShapeDtypeStruct((1, 1), F32),
                   jax.ShapeDtypeStruct((1, 1), F32)),
    )(u, ipos, ineg)

    loss = (bpr_s[0, 0] / B + 1e-4 * (reg_s[0, 0] / B)
            + 0.1 * (align[0, 0] / (NI * D)))
    return loss, u_final, i_final
